# baseline (speedup 1.0000x reference)
import functools

import jax
import jax.numpy as jnp
from jax import lax
from jax.experimental import pallas as pl
from jax.experimental.pallas import tpu as pltpu
from jax.experimental.pallas import tpu_sc as plsc

B = 4096
D = 1024
EPS = 1e-12


BM_PREP = 512


def _prep_body(x_ref, xn_ref, sq_ref):
    x = x_ref[...]
    n2 = jnp.sum(x * x, axis=1, keepdims=True)
    n = jnp.sqrt(n2)
    xn = x / jnp.maximum(n, EPS)
    xn_ref[...] = xn
    sq_ref[...] = jnp.sum(xn * xn, axis=1, keepdims=True)


def _prep(x):
    return pl.pallas_call(
        _prep_body,
        grid=(B // BM_PREP,),
        in_specs=[pl.BlockSpec((BM_PREP, D), lambda i: (i, 0))],
        out_specs=[
            pl.BlockSpec((BM_PREP, D), lambda i: (i, 0)),
            pl.BlockSpec((BM_PREP, 1), lambda i: (i, 0)),
        ],
        out_shape=[
            jax.ShapeDtypeStruct((B, D), jnp.float32),
            jax.ShapeDtypeStruct((B, 1), jnp.float32),
        ],
    )(x)


BM = 512
BN = 512
NJ = B // BN


def _argmax_body(a_ref, b_ref, sqr_ref, sqc_ref, idx_ref, m_sc, i_sc):
    j = pl.program_id(1)

    @pl.when(j == 0)
    def _init():
        m_sc[...] = jnp.full((BM, 1), -jnp.inf, jnp.float32)
        i_sc[...] = jnp.zeros((BM, 1), jnp.int32)

    g = lax.dot_general(
        a_ref[...], b_ref[...],
        dimension_numbers=(((1,), (1,)), ((), ())),
        preferred_element_type=jnp.float32,
    )
    d2 = (sqr_ref[...] + sqc_ref[...]) - 2.0 * g
    d = jnp.sqrt(jnp.maximum(d2, 0.0))
    m = jnp.max(d, axis=1, keepdims=True)
    col = lax.broadcasted_iota(jnp.int32, (BM, BN), 1)
    big = jnp.int32(2 * B)
    loc = jnp.min(jnp.where(d == m, col, big), axis=1, keepdims=True)
    gidx = loc + j * BN
    better = m > m_sc[...]
    i_sc[...] = jnp.where(better, gidx, i_sc[...])
    m_sc[...] = jnp.where(better, m, m_sc[...])

    @pl.when(j == NJ - 1)
    def _done():
        idx_ref[...] = i_sc[...]


def _argmax(xn, sqr, sqc):
    return pl.pallas_call(
        _argmax_body,
        grid=(B // BM, NJ),
        in_specs=[
            pl.BlockSpec((BM, D), lambda i, j: (i, 0)),
            pl.BlockSpec((BN, D), lambda i, j: (j, 0)),
            pl.BlockSpec((BM, 1), lambda i, j: (i, 0)),
            pl.BlockSpec((1, BN), lambda i, j: (0, j)),
        ],
        out_specs=pl.BlockSpec((BM, 1), lambda i, j: (i, 0)),
        out_shape=jax.ShapeDtypeStruct((B, 1), jnp.int32),
        scratch_shapes=[
            pltpu.VMEM((BM, 1), jnp.float32),
            pltpu.VMEM((BM, 1), jnp.int32),
        ],
        compiler_params=pltpu.CompilerParams(
            dimension_semantics=("arbitrary", "arbitrary"),
        ),
    )(xn, xn, sqr, sqc)


_NC = 2
_NS = 16
_NW = _NC * _NS
_BPW = B // _NW
_CH = 32


def _sc_gather_body(table_hbm, idx_hbm, out_hbm, idx_v, rows_v, sem):
    wid = lax.axis_index("s") * _NC + lax.axis_index("c")
    base = wid * _BPW
    pltpu.sync_copy(idx_hbm.at[pl.ds(base, _BPW)], idx_v)

    def chunk(c, _):
        off = c * _CH
        pltpu.async_copy(
            table_hbm.at[idx_v.at[pl.ds(off, _CH)]],
            rows_v, sem).wait()
        pltpu.sync_copy(rows_v, out_hbm.at[pl.ds(base + off, _CH)])
        return _

    lax.fori_loop(0, _BPW // _CH, chunk, 0)


def _sc_gather(x, idx):
    mesh = plsc.VectorSubcoreMesh(core_axis_name="c", subcore_axis_name="s")
    k = functools.partial(
        pl.kernel,
        mesh=mesh,
        out_type=jax.ShapeDtypeStruct((B, D), jnp.float32),
        scratch_types=[
            pltpu.VMEM((_BPW,), jnp.int32),
            pltpu.VMEM((_CH, D), jnp.float32),
            pltpu.SemaphoreType.DMA,
        ],
    )(_sc_gather_body)
    return k(x, idx)


BM_EP = 512


def _colsum_body(xg_ref, s_ref):
    @pl.when(pl.program_id(0) == 0)
    def _init():
        s_ref[...] = jnp.zeros((1, D), jnp.float32)

    s_ref[...] += jnp.sum(xg_ref[...], axis=0, keepdims=True)


def _colsum(xg):
    return pl.pallas_call(
        _colsum_body,
        grid=(B // BM_EP,),
        in_specs=[pl.BlockSpec((BM_EP, D), lambda i: (i, 0))],
        out_specs=pl.BlockSpec((1, D), lambda i: (0, 0)),
        out_shape=jax.ShapeDtypeStruct((1, D), jnp.float32),
        compiler_params=pltpu.CompilerParams(
            dimension_semantics=("arbitrary",),
        ),
    )(xg)


def _axpy_body(x_ref, xg_ref, s_ref, o_ref):
    kf = jnp.float32(B)
    inv = jnp.float32(1.0 / (B + 1.0))
    o_ref[...] = x_ref[...] + (s_ref[...] - kf * xg_ref[...]) * inv


def _axpy(x, xg, s):
    return pl.pallas_call(
        _axpy_body,
        grid=(B // BM_EP,),
        in_specs=[
            pl.BlockSpec((BM_EP, D), lambda i: (i, 0)),
            pl.BlockSpec((BM_EP, D), lambda i: (i, 0)),
            pl.BlockSpec((1, D), lambda i: (0, 0)),
        ],
        out_specs=pl.BlockSpec((BM_EP, D), lambda i: (i, 0)),
        out_shape=jax.ShapeDtypeStruct((B, D), jnp.float32),
    )(x, xg, s)


def kernel(x):
    xn, sq = _prep(x)
    sqc = sq.reshape(1, B)
    idx2 = _argmax(xn, sq, sqc)
    idx = idx2.reshape(B)
    xg = _sc_gather(x, idx)
    s = _colsum(xg)
    return _axpy(x, xg, s)


# speedup vs baseline: 1.2487x; 1.2487x over previous
import functools

import jax
import jax.numpy as jnp
from jax import lax
from jax.experimental import pallas as pl
from jax.experimental.pallas import tpu as pltpu
from jax.experimental.pallas import tpu_sc as plsc

B = 4096
D = 1024
EPS = 1e-12


BM = 256
NJ = B // BM


def _fused_body(x_ref, idx_ref, xn_sc, sqc_sc):
    p = pl.program_id(0)
    j = pl.program_id(1)

    @pl.when(p == 0)
    def _norm():
        xc = x_ref[...]
        n2 = jnp.sum(xc * xc, axis=1, keepdims=True)
        xn = xc / jnp.maximum(jnp.sqrt(n2), EPS)
        xn_sc[pl.ds(j * BM, BM), :] = xn
        sq = jnp.sum(xn * xn, axis=1, keepdims=True)
        sqc_sc[:, pl.ds(j * BM, BM)] = sq.T

    @pl.when(p == 1)
    def _dist():
        a = xn_sc[pl.ds(j * BM, BM), :]
        sqr = jnp.sum(a * a, axis=1, keepdims=True)
        g2 = lax.dot_general(
            a * (-2.0), xn_sc[...],
            dimension_numbers=(((1,), (1,)), ((), ())),
            preferred_element_type=jnp.float32,
        )
        d2 = (sqr + sqc_sc[...]) + g2
        d = jnp.sqrt(jnp.maximum(d2, 0.0))
        m = jnp.max(d, axis=1, keepdims=True)
        col = lax.broadcasted_iota(jnp.int32, (BM, B), 1)
        big = jnp.int32(2 * B)
        idx_ref[...] = jnp.min(
            jnp.where(d == m, col, big), axis=1, keepdims=True)


def _argmax_fused(x):
    return pl.pallas_call(
        _fused_body,
        grid=(2, NJ),
        in_specs=[pl.BlockSpec((BM, D), lambda p, j: (j, 0))],
        out_specs=pl.BlockSpec((BM, 1), lambda p, j: (j, 0)),
        out_shape=jax.ShapeDtypeStruct((B, 1), jnp.int32),
        scratch_shapes=[
            pltpu.VMEM((B, D), jnp.float32),
            pltpu.VMEM((1, B), jnp.float32),
        ],
        compiler_params=pltpu.CompilerParams(
            dimension_semantics=("arbitrary", "arbitrary"),
        ),
    )(x)


_NC = 2
_NS = 16
_NW = _NC * _NS
_BPW = B // _NW
_CH = 32


def _sc_gather_body(table_hbm, idx_hbm, out_hbm, idx_v, rows_v, sem):
    wid = lax.axis_index("s") * _NC + lax.axis_index("c")
    base = wid * _BPW
    pltpu.sync_copy(idx_hbm.at[pl.ds(base, _BPW)], idx_v)

    def chunk(c, _):
        off = c * _CH
        pltpu.async_copy(
            table_hbm.at[idx_v.at[pl.ds(off, _CH)]],
            rows_v, sem).wait()
        pltpu.sync_copy(rows_v, out_hbm.at[pl.ds(base + off, _CH)])
        return _

    lax.fori_loop(0, _BPW // _CH, chunk, 0)


def _sc_gather(x, idx):
    mesh = plsc.VectorSubcoreMesh(core_axis_name="c", subcore_axis_name="s")
    k = functools.partial(
        pl.kernel,
        mesh=mesh,
        out_type=jax.ShapeDtypeStruct((B, D), jnp.float32),
        scratch_types=[
            pltpu.VMEM((_BPW,), jnp.int32),
            pltpu.VMEM((_CH, D), jnp.float32),
            pltpu.SemaphoreType.DMA,
        ],
    )(_sc_gather_body)
    return k(x, idx)


BM_EP = 512


def _colsum_body(xg_ref, s_ref):
    @pl.when(pl.program_id(0) == 0)
    def _init():
        s_ref[...] = jnp.zeros((1, D), jnp.float32)

    s_ref[...] += jnp.sum(xg_ref[...], axis=0, keepdims=True)


def _colsum(xg):
    return pl.pallas_call(
        _colsum_body,
        grid=(B // BM_EP,),
        in_specs=[pl.BlockSpec((BM_EP, D), lambda i: (i, 0))],
        out_specs=pl.BlockSpec((1, D), lambda i: (0, 0)),
        out_shape=jax.ShapeDtypeStruct((1, D), jnp.float32),
        compiler_params=pltpu.CompilerParams(
            dimension_semantics=("arbitrary",),
        ),
    )(xg)


def _axpy_body(x_ref, xg_ref, s_ref, o_ref):
    kf = jnp.float32(B)
    inv = jnp.float32(1.0 / (B + 1.0))
    o_ref[...] = x_ref[...] + (s_ref[...] - kf * xg_ref[...]) * inv


def _axpy(x, xg, s):
    return pl.pallas_call(
        _axpy_body,
        grid=(B // BM_EP,),
        in_specs=[
            pl.BlockSpec((BM_EP, D), lambda i: (i, 0)),
            pl.BlockSpec((BM_EP, D), lambda i: (i, 0)),
            pl.BlockSpec((1, D), lambda i: (0, 0)),
        ],
        out_specs=pl.BlockSpec((BM_EP, D), lambda i: (i, 0)),
        out_shape=jax.ShapeDtypeStruct((B, D), jnp.float32),
    )(x, xg, s)


def kernel(x):
    idx2 = _argmax_fused(x)
    idx = idx2.reshape(B)
    xg = _sc_gather(x, idx)
    s = _colsum(xg)
    return _axpy(x, xg, s)
